# baseline (speedup 1.0000x reference)
import functools

import jax
import jax.numpy as jnp
from jax.experimental import pallas as pl

B, M, D, E, T, TOPK = 4096, 6, 128, 6, 60, 2
N = B * M
OUT_W = 128


def _gelu(x):
    return 0.5 * x * (1.0 + jax.lax.erf(x * 0.7071067811865476))


def _moe_kernel(x_ref, wr1_ref, br1_ref, wr2_ref, br2_ref, wr3_ref, br3_ref,
                wt1_ref, bt1_ref, wt2_ref, bt2_ref, wt3_ref, bt3_ref,
                ws1_ref, bs1_ref, ws2_ref, bs2_ref, ws3t_ref, bs3_ref,
                traj_ref, rp_ref, sc_ref, aux_ref):
    f32 = jnp.float32
    x3 = x_ref[...]
    blk = x3.shape[0] * M
    x = x3.reshape(blk, D)

    h = _gelu(jnp.dot(x, wr1_ref[...], preferred_element_type=f32) + br1_ref[...])
    h = _gelu(jnp.dot(h, wr2_ref[...], preferred_element_type=f32) + br2_ref[...])
    logits = jnp.dot(h, wr3_ref[...], preferred_element_type=f32) + br3_ref[...]

    idx = jax.lax.broadcasted_iota(jnp.int32, (blk, E), 1)
    m1 = jnp.max(logits, axis=1, keepdims=True)
    i1 = jnp.min(jnp.where(logits == m1, idx, E), axis=1, keepdims=True)
    masked = jnp.where(idx == i1, -jnp.inf, logits)
    m2 = jnp.max(masked, axis=1, keepdims=True)
    i2 = jnp.min(jnp.where(masked == m2, idx, E), axis=1, keepdims=True)
    el = jnp.exp(logits - m1)
    rp = el / jnp.sum(el, axis=1, keepdims=True)
    sel = (idx == i1) | (idx == i2)
    wsel = jnp.where(sel, el, 0.0)
    sw = wsel / jnp.sum(wsel, axis=1, keepdims=True)

    acc_traj = jnp.zeros((blk, T * 2), f32)
    acc_sc = jnp.zeros((blk, 1), f32)
    for e in range(E):
        g = sw[:, e:e + 1]
        h1 = _gelu(jnp.dot(x, wt1_ref[e], preferred_element_type=f32) + bt1_ref[e:e + 1, :])
        h2 = _gelu(jnp.dot(h1, wt2_ref[e], preferred_element_type=f32) + bt2_ref[e:e + 1, :])
        tr = jnp.dot(h2, wt3_ref[e], preferred_element_type=f32) + bt3_ref[e:e + 1, :]
        acc_traj = acc_traj + g * tr
        s1 = _gelu(jnp.dot(x, ws1_ref[e], preferred_element_type=f32) + bs1_ref[e:e + 1, :])
        s2 = _gelu(jnp.dot(s1, ws2_ref[e], preferred_element_type=f32) + bs2_ref[e:e + 1, :])
        sc = jnp.sum(s2 * ws3t_ref[e], axis=1, keepdims=True) + bs3_ref[e:e + 1, :]
        acc_sc = acc_sc + g * sc

    bb = blk // M
    traj_ref[...] = acc_traj
    rp_ref[...] = rp.reshape(bb, M, E)
    sc_ref[...] = acc_sc.reshape(bb, M)

    @pl.when(pl.program_id(0) == 0)
    def _init():
        aux_ref[...] = jnp.zeros_like(aux_ref)

    rp_sum = jnp.sum(rp, axis=0, keepdims=True)
    aux_ref[0:1, 0:E] = aux_ref[0:1, 0:E] + rp_sum


def _run(mode_features, Wr1, br1, Wr2, br2, Wr3, br3, Wt1, bt1, Wt2, bt2,
         Wt3, bt3, Ws1, bs1, Ws2, bs2, Ws3, bs3, bb=256):
    blk = bb * M
    ws3t = jnp.transpose(Ws3, (0, 2, 1))
    full = lambda a: pl.BlockSpec(a.shape, lambda i: (0,) * a.ndim)
    args = (mode_features, Wr1, br1.reshape(1, 256), Wr2, br2.reshape(1, 128),
            Wr3, br3.reshape(1, E), Wt1, bt1, Wt2, bt2, Wt3, bt3,
            Ws1, bs1, Ws2, bs2, ws3t, bs3)
    in_specs = [pl.BlockSpec((bb, M, D), lambda i: (i, 0, 0))]
    in_specs += [full(a) for a in args[1:]]
    traj, rp, scores, aux = pl.pallas_call(
        _moe_kernel,
        grid=(B // bb,),
        in_specs=in_specs,
        out_specs=[pl.BlockSpec((blk, T * 2), lambda i: (i, 0)),
                   pl.BlockSpec((bb, M, E), lambda i: (i, 0, 0)),
                   pl.BlockSpec((bb, M), lambda i: (i, 0)),
                   pl.BlockSpec((8, 128), lambda i: (0, 0))],
        out_shape=[jax.ShapeDtypeStruct((N, T * 2), jnp.float32),
                   jax.ShapeDtypeStruct((B, M, E), jnp.float32),
                   jax.ShapeDtypeStruct((B, M), jnp.float32),
                   jax.ShapeDtypeStruct((8, 128), jnp.float32)],
    )(*args)
    trajectories = traj.reshape(B, M, T, 2)
    avg = aux[0, :E] / N
    aux_loss = E * jnp.sum(avg * avg)
    return trajectories, scores, aux_loss, rp


_jitted = jax.jit(_run, static_argnames=("bb",))


def kernel(mode_features, Wr1, br1, Wr2, br2, Wr3, br3, Wt1, bt1, Wt2, bt2,
           Wt3, bt3, Ws1, bs1, Ws2, bs2, Ws3, bs3):
    return _jitted(mode_features, Wr1, br1, Wr2, br2, Wr3, br3, Wt1, bt1,
                   Wt2, bt2, Wt3, bt3, Ws1, bs1, Ws2, bs2, Ws3, bs3)


# speedup vs baseline: 1.9942x; 1.9942x over previous
import functools

import jax
import jax.numpy as jnp
from jax.experimental import pallas as pl

B, M, D, E, T, TOPK = 4096, 6, 128, 6, 60, 2
N = B * M


def _gelu(x):
    return 0.5 * x * (1.0 + jax.lax.erf(x * 0.7071067811865476))


def _dotT(w, h):
    return jax.lax.dot_general(w, h, (((0,), (0,)), ((), ())),
                               preferred_element_type=jnp.float32)


def _moe_kernel(x_ref, wr1_ref, br1_ref, wr2_ref, br2_ref, wr3_ref, br3_ref,
                wt1_ref, bt1_ref, wt2_ref, bt2_ref, wt3_ref, bt3_ref,
                ws1_ref, bs1_ref, ws2_ref, bs2_ref, ws3_ref, bs3_ref,
                traj_ref, rp_ref, sc_ref, aux_ref):
    f32 = jnp.float32
    m = pl.program_id(1)
    x = x_ref[0]
    bb = x.shape[0]
    xt = x.T

    h = _gelu(_dotT(wr1_ref[...], xt) + br1_ref[...])
    h = _gelu(_dotT(wr2_ref[...], h) + br2_ref[...])
    lg = _dotT(wr3_ref[...], h) + br3_ref[...]

    idx = jax.lax.broadcasted_iota(jnp.int32, (E, bb), 0)
    m1 = jnp.max(lg, axis=0, keepdims=True)
    i1 = jnp.min(jnp.where(lg == m1, idx, E), axis=0, keepdims=True)
    masked = jnp.where(idx == i1, -jnp.inf, lg)
    m2 = jnp.max(masked, axis=0, keepdims=True)
    i2 = jnp.min(jnp.where(masked == m2, idx, E), axis=0, keepdims=True)
    el = jnp.exp(lg - m1)
    rp = el / jnp.sum(el, axis=0, keepdims=True)
    sel = (idx == i1) | (idx == i2)
    wsel = jnp.where(sel, el, 0.0)
    sw = wsel / jnp.sum(wsel, axis=0, keepdims=True)

    acc_traj = jnp.zeros((T * 2, bb), f32)
    acc_sc = jnp.zeros((1, bb), f32)
    for e in range(E):
        g = sw[e:e + 1, :]
        h1 = _gelu(_dotT(wt1_ref[e], xt) + bt1_ref[e])
        h2 = _gelu(_dotT(wt2_ref[e], h1) + bt2_ref[e])
        tr = _dotT(wt3_ref[e], h2) + bt3_ref[e]
        acc_traj = acc_traj + g * tr
        s1 = _gelu(_dotT(ws1_ref[e], xt) + bs1_ref[e])
        s2 = _gelu(_dotT(ws2_ref[e], s1) + bs2_ref[e])
        sc = jnp.sum(s2 * ws3_ref[e], axis=0, keepdims=True) + bs3_ref[e]
        acc_sc = acc_sc + g * sc

    traj_ref[...] = acc_traj
    rp_ref[0] = rp
    sc_ref[pl.ds(m, 1), :] = acc_sc

    @pl.when((pl.program_id(0) == 0) & (m == 0))
    def _init():
        aux_ref[...] = jnp.zeros_like(aux_ref)

    aux_ref[0:E, 0:1] = aux_ref[0:E, 0:1] + jnp.sum(rp, axis=1, keepdims=True)


@functools.partial(jax.jit, static_argnames=("bb",))
def _run(mode_features, Wr1, br1, Wr2, br2, Wr3, br3, Wt1, bt1, Wt2, bt2,
         Wt3, bt3, Ws1, bs1, Ws2, bs2, Ws3, bs3, bb=512):
    f32 = jnp.float32
    xT = jnp.transpose(mode_features, (1, 0, 2))
    col = lambda b: b.reshape(-1, 1)
    ecol = lambda b: b[:, :, None]
    full = lambda a: pl.BlockSpec(a.shape, lambda i, m: (0,) * a.ndim)
    args = (xT, Wr1, col(br1), Wr2, col(br2), Wr3, col(br3),
            Wt1, ecol(bt1), Wt2, ecol(bt2), Wt3, ecol(bt3),
            Ws1, ecol(bs1), Ws2, ecol(bs2), Ws3, ecol(bs3))
    in_specs = [pl.BlockSpec((1, bb, D), lambda i, m: (m, i, 0))]
    in_specs += [full(a) for a in args[1:]]
    P, rp_p, sc_p, aux = pl.pallas_call(
        _moe_kernel,
        grid=(B // bb, M),
        in_specs=in_specs,
        out_specs=[pl.BlockSpec((T * 2, bb), lambda i, m: (m, i)),
                   pl.BlockSpec((1, E, bb), lambda i, m: (m, 0, i)),
                   pl.BlockSpec((M, bb), lambda i, m: (0, i)),
                   pl.BlockSpec((8, 128), lambda i, m: (0, 0))],
        out_shape=[jax.ShapeDtypeStruct((M * T * 2, B), f32),
                   jax.ShapeDtypeStruct((M, E, B), f32),
                   jax.ShapeDtypeStruct((M, B), f32),
                   jax.ShapeDtypeStruct((8, 128), f32)],
    )(*args)
    trajectories = jnp.transpose(P.reshape(M, T, 2, B), (3, 0, 1, 2))
    rp = jnp.transpose(rp_p, (2, 0, 1))
    scores = jnp.transpose(sc_p, (1, 0))
    avg = aux[:E, 0] / N
    aux_loss = E * jnp.sum(avg * avg)
    return trajectories, scores, aux_loss, rp


def kernel(mode_features, Wr1, br1, Wr2, br2, Wr3, br3, Wt1, bt1, Wt2, bt2,
           Wt3, bt3, Ws1, bs1, Ws2, bs2, Ws3, bs3):
    return _run(mode_features, Wr1, br1, Wr2, br2, Wr3, br3, Wt1, bt1,
                Wt2, bt2, Wt3, bt3, Ws1, bs1, Ws2, bs2, Ws3, bs3)


# speedup vs baseline: 3.2188x; 1.6141x over previous
import functools

import jax
import jax.numpy as jnp
from jax.experimental import pallas as pl

B, M, D, E, T, TOPK = 4096, 6, 128, 6, 60, 2
N = B * M


def _gelu(x):
    return 0.5 * x * (1.0 + jax.lax.erf(x * 0.7071067811865476))


def _dotT(w, h):
    return jax.lax.dot_general(w, h, (((0,), (0,)), ((), ())),
                               preferred_element_type=jnp.float32)


def _moe_kernel(x_ref, wr1_ref, br1_ref, wr2_ref, br2_ref, wr3_ref, br3_ref,
                wt1_ref, bt1_ref, wt2_ref, bt2_ref, wt3_ref, bt3_ref,
                ws1_ref, bs1_ref, ws2_ref, bs2_ref, ws3_ref, bs3_ref,
                traj_ref, rp_ref, sc_ref, aux_ref):
    f32 = jnp.float32
    m = pl.program_id(1)
    x = x_ref[0]
    bb = x.shape[0]
    xt = x.T

    h = _gelu(_dotT(wr1_ref[...], xt) + br1_ref[...])
    h = _gelu(_dotT(wr2_ref[...], h) + br2_ref[...])
    lg = _dotT(wr3_ref[...], h) + br3_ref[...]

    idx = jax.lax.broadcasted_iota(jnp.int32, (E, bb), 0)
    m1 = jnp.max(lg, axis=0, keepdims=True)
    i1 = jnp.min(jnp.where(lg == m1, idx, E), axis=0, keepdims=True)
    masked = jnp.where(idx == i1, -jnp.inf, lg)
    m2 = jnp.max(masked, axis=0, keepdims=True)
    i2 = jnp.min(jnp.where(masked == m2, idx, E), axis=0, keepdims=True)
    el = jnp.exp(lg - m1)
    rp = el / jnp.sum(el, axis=0, keepdims=True)
    sel = (idx == i1) | (idx == i2)
    wsel = jnp.where(sel, el, 0.0)
    sw = wsel / jnp.sum(wsel, axis=0, keepdims=True)

    acc_traj = jnp.zeros((T * 2, bb), f32)
    acc_sc = jnp.zeros((1, bb), f32)
    for e in range(E):
        g = sw[e:e + 1, :]
        h1 = _gelu(_dotT(wt1_ref[e], xt) + bt1_ref[e])
        h2 = _gelu(_dotT(wt2_ref[e], h1) + bt2_ref[e])
        tr = _dotT(wt3_ref[e], h2) + bt3_ref[e]
        acc_traj = acc_traj + g * tr
        s1 = _gelu(_dotT(ws1_ref[e], xt) + bs1_ref[e])
        s2 = _gelu(_dotT(ws2_ref[e], s1) + bs2_ref[e])
        sc = jnp.sum(s2 * ws3_ref[e], axis=0, keepdims=True) + bs3_ref[e]
        acc_sc = acc_sc + g * sc

    traj_ref[...] = acc_traj
    rp_ref[0] = rp
    sc_ref[pl.ds(m, 1), :] = acc_sc

    @pl.when((pl.program_id(0) == 0) & (m == 0))
    def _init():
        aux_ref[...] = jnp.zeros_like(aux_ref)

    aux_ref[0:E, 0:1] = aux_ref[0:E, 0:1] + jnp.sum(rp, axis=1, keepdims=True)


@functools.partial(jax.jit, static_argnames=("bb",))
def _run(mode_features, Wr1, br1, Wr2, br2, Wr3, br3, Wt1, bt1, Wt2, bt2,
         Wt3, bt3, Ws1, bs1, Ws2, bs2, Ws3, bs3, bb=2048):
    f32 = jnp.float32
    xT = jnp.transpose(mode_features, (1, 0, 2))
    col = lambda b: b.reshape(-1, 1)
    ecol = lambda b: b[:, :, None]
    full = lambda a: pl.BlockSpec(a.shape, lambda i, m: (0,) * a.ndim)
    args = (xT, Wr1, col(br1), Wr2, col(br2), Wr3, col(br3),
            Wt1, ecol(bt1), Wt2, ecol(bt2), Wt3, ecol(bt3),
            Ws1, ecol(bs1), Ws2, ecol(bs2), Ws3, ecol(bs3))
    in_specs = [pl.BlockSpec((1, bb, D), lambda i, m: (m, i, 0))]
    in_specs += [full(a) for a in args[1:]]
    P, rp_p, sc_p, aux = pl.pallas_call(
        _moe_kernel,
        grid=(B // bb, M),
        in_specs=in_specs,
        out_specs=[pl.BlockSpec((T * 2, bb), lambda i, m: (m, i)),
                   pl.BlockSpec((1, E, bb), lambda i, m: (m, 0, i)),
                   pl.BlockSpec((M, bb), lambda i, m: (0, i)),
                   pl.BlockSpec((8, 128), lambda i, m: (0, 0))],
        out_shape=[jax.ShapeDtypeStruct((M * T * 2, B), f32),
                   jax.ShapeDtypeStruct((M, E, B), f32),
                   jax.ShapeDtypeStruct((M, B), f32),
                   jax.ShapeDtypeStruct((8, 128), f32)],
    )(*args)
    trajectories = jnp.transpose(P.reshape(M, T, 2, B), (3, 0, 1, 2))
    rp = jnp.transpose(rp_p, (2, 0, 1))
    scores = jnp.transpose(sc_p, (1, 0))
    avg = aux[:E, 0] / N
    aux_loss = E * jnp.sum(avg * avg)
    return trajectories, scores, aux_loss, rp


def kernel(mode_features, Wr1, br1, Wr2, br2, Wr3, br3, Wt1, bt1, Wt2, bt2,
           Wt3, bt3, Ws1, bs1, Ws2, bs2, Ws3, bs3):
    return _run(mode_features, Wr1, br1, Wr2, br2, Wr3, br3, Wt1, bt1,
                Wt2, bt2, Wt3, bt3, Ws1, bs1, Ws2, bs2, Ws3, bs3)


# speedup vs baseline: 3.3739x; 1.0482x over previous
import functools

import jax
import jax.numpy as jnp
from jax.experimental import pallas as pl

B, M, D, E, T, TOPK = 4096, 6, 128, 6, 60, 2
N = B * M


def _gelu(x):
    return 0.5 * x * (1.0 + jax.lax.erf(x * 0.7071067811865476))


def _dotT(w, h):
    return jax.lax.dot_general(w, h, (((0,), (0,)), ((), ())),
                               preferred_element_type=jnp.float32)


def _moe_kernel(x_ref, wr1_ref, br1_ref, wr2_ref, br2_ref, wr3_ref, br3_ref,
                wt1_ref, bt1_ref, wt2_ref, bt2_ref, wt3_ref, bt3_ref,
                ws1_ref, bs1_ref, ws2_ref, bs2_ref, ws3_ref, bs3_ref,
                traj_ref, rp_ref, sc_ref, aux_ref):
    f32 = jnp.float32
    m = pl.program_id(1)
    x = x_ref[0]
    bb = x.shape[0]
    xt = x.T

    h = _gelu(_dotT(wr1_ref[...], xt) + br1_ref[...])
    h = _gelu(_dotT(wr2_ref[...], h) + br2_ref[...])
    lg = _dotT(wr3_ref[...], h) + br3_ref[...]

    idx = jax.lax.broadcasted_iota(jnp.int32, (E, bb), 0)
    m1 = jnp.max(lg, axis=0, keepdims=True)
    i1 = jnp.min(jnp.where(lg == m1, idx, E), axis=0, keepdims=True)
    masked = jnp.where(idx == i1, -jnp.inf, lg)
    m2 = jnp.max(masked, axis=0, keepdims=True)
    i2 = jnp.min(jnp.where(masked == m2, idx, E), axis=0, keepdims=True)
    el = jnp.exp(lg - m1)
    rp = el / jnp.sum(el, axis=0, keepdims=True)
    sel = (idx == i1) | (idx == i2)
    wsel = jnp.where(sel, el, 0.0)
    sw = wsel / jnp.sum(wsel, axis=0, keepdims=True)

    acc_traj = jnp.zeros((T * 2, bb), f32)
    acc_sc = jnp.zeros((1, bb), f32)
    for e in range(E):
        g = sw[e:e + 1, :]
        h1 = _gelu(_dotT(wt1_ref[e], xt) + bt1_ref[e])
        h2 = _gelu(_dotT(wt2_ref[e], h1) + bt2_ref[e])
        tr = _dotT(wt3_ref[e], h2) + bt3_ref[e]
        acc_traj = acc_traj + g * tr
        s1 = _gelu(_dotT(ws1_ref[e], xt) + bs1_ref[e])
        s2 = _gelu(_dotT(ws2_ref[e], s1) + bs2_ref[e])
        sc = jnp.sum(s2 * ws3_ref[e], axis=0, keepdims=True) + bs3_ref[e]
        acc_sc = acc_sc + g * sc

    traj_ref[...] = acc_traj
    rp_ref[0] = rp
    sc_ref[pl.ds(m, 1), :] = acc_sc

    @pl.when((pl.program_id(0) == 0) & (m == 0))
    def _init():
        aux_ref[...] = jnp.zeros_like(aux_ref)

    aux_ref[0:E, 0:1] = aux_ref[0:E, 0:1] + jnp.sum(rp, axis=1, keepdims=True)


@functools.partial(jax.jit, static_argnames=("bb",))
def _run(mode_features, Wr1, br1, Wr2, br2, Wr3, br3, Wt1, bt1, Wt2, bt2,
         Wt3, bt3, Ws1, bs1, Ws2, bs2, Ws3, bs3, bb=4096):
    f32 = jnp.float32
    xT = jnp.transpose(mode_features, (1, 0, 2))
    col = lambda b: b.reshape(-1, 1)
    ecol = lambda b: b[:, :, None]
    full = lambda a: pl.BlockSpec(a.shape, lambda i, m: (0,) * a.ndim)
    args = (xT, Wr1, col(br1), Wr2, col(br2), Wr3, col(br3),
            Wt1, ecol(bt1), Wt2, ecol(bt2), Wt3, ecol(bt3),
            Ws1, ecol(bs1), Ws2, ecol(bs2), Ws3, ecol(bs3))
    in_specs = [pl.BlockSpec((1, bb, D), lambda i, m: (m, i, 0))]
    in_specs += [full(a) for a in args[1:]]
    P, rp_p, sc_p, aux = pl.pallas_call(
        _moe_kernel,
        grid=(B // bb, M),
        in_specs=in_specs,
        out_specs=[pl.BlockSpec((T * 2, bb), lambda i, m: (m, i)),
                   pl.BlockSpec((1, E, bb), lambda i, m: (m, 0, i)),
                   pl.BlockSpec((M, bb), lambda i, m: (0, i)),
                   pl.BlockSpec((8, 128), lambda i, m: (0, 0))],
        out_shape=[jax.ShapeDtypeStruct((M * T * 2, B), f32),
                   jax.ShapeDtypeStruct((M, E, B), f32),
                   jax.ShapeDtypeStruct((M, B), f32),
                   jax.ShapeDtypeStruct((8, 128), f32)],
    )(*args)
    trajectories = jnp.transpose(P.reshape(M, T, 2, B), (3, 0, 1, 2))
    rp = jnp.transpose(rp_p, (2, 0, 1))
    scores = jnp.transpose(sc_p, (1, 0))
    avg = aux[:E, 0] / N
    aux_loss = E * jnp.sum(avg * avg)
    return trajectories, scores, aux_loss, rp


def kernel(mode_features, Wr1, br1, Wr2, br2, Wr3, br3, Wt1, bt1, Wt2, bt2,
           Wt3, bt3, Ws1, bs1, Ws2, bs2, Ws3, bs3):
    return _run(mode_features, Wr1, br1, Wr2, br2, Wr3, br3, Wt1, bt1,
                Wt2, bt2, Wt3, bt3, Ws1, bs1, Ws2, bs2, Ws3, bs3)
